# baseline (speedup 1.0000x reference)
import functools

import jax
import jax.numpy as jnp
from jax import lax
from jax.experimental import pallas as pl
from jax.experimental.pallas import tpu as pltpu
from jax.experimental.pallas import tpu_sc as plsc

_N = 10000
_E = 320000
_NFEAT = 128
_NHID = 128
_NCLASS = 64

_NC = 2
_NS = 16
_NT = _NC * _NS
_CH = 128
_CPT = 80
_CHUNKS_PAD = _CPT * _NT
_E_PAD = _CHUNKS_PAD * _CH
_ACH = 64
_ACPT = _E_PAD // (_NT * _ACH)
_NBUF = 4
_ACC_ROWS = 10240
_RPT = _ACC_ROWS // _NS
_ACC_A = 10112
_RPA = _ACC_A // _NS
_HALF = _CPT // 2

@functools.cache
def _mesh():
    return plsc.VectorSubcoreMesh(
        core_axis_name="c", subcore_axis_name="s", num_cores=_NC, num_subcores=_NS
    )


def _deg_body(dst_hbm, out_hbm, idx_v, ones_v, zero_v, acc):
    cid = lax.axis_index("c")
    sid = lax.axis_index("s")
    tid = cid * _NS + sid
    for i in range(_CH // 16):
        ones_v[pl.ds(i * 16, 16)] = jnp.ones((16,), jnp.float32)
    for i in range(_RPT // 16):
        zero_v[pl.ds(i * 16, 16)] = jnp.zeros((16,), jnp.float32)
    pltpu.sync_copy(zero_v, acc.at[pl.ds(sid * _RPT, _RPT)])
    plsc.subcore_barrier()
    pltpu.sync_copy(dst_hbm.at[pl.ds(tid * _CPT, _CPT)], idx_v)

    def body(i, carry):
        pltpu.sync_copy(ones_v, acc.at[idx_v.at[i]], add=True)
        return carry

    lax.fori_loop(0, _CPT, body, 0)
    plsc.subcore_barrier()
    pltpu.sync_copy(
        acc.at[pl.ds(sid * _RPT, _RPT)],
        out_hbm.at[pl.ds(cid * _ACC_ROWS + sid * _RPT, _RPT)],
    )


@functools.cache
def _deg_call():
    return functools.partial(
        pl.kernel,
        out_type=jax.ShapeDtypeStruct((_NC * _ACC_ROWS,), jnp.float32),
        mesh=_mesh(),
        scratch_types=[
            pltpu.VMEM((_CPT, _CH), jnp.int32),
            pltpu.VMEM((_CH,), jnp.float32),
            pltpu.VMEM((_RPT,), jnp.float32),
            pltpu.VMEM_SHARED((_ACC_ROWS,), jnp.float32),
        ],
    )(_deg_body)


def _agg_body(f, y_hbm, src_hbm, dst_hbm, out_hbm, srcv, dstv, rows, acc, sems,
              ssems):
    cid = lax.axis_index("c")
    sid = lax.axis_index("s")
    tid = cid * _NS + sid

    def zfill(r, carry):
        for c in range(f // 16):
            rows[0][r, pl.ds(c * 16, 16)] = jnp.zeros((16,), jnp.float32)
        return carry

    lax.fori_loop(0, _ACH, zfill, 0)
    for k in range(_RPA // _ACH):
        pltpu.sync_copy(rows[0], acc.at[pl.ds(sid * _RPA + k * _ACH, _ACH)])
    rem = _RPA % _ACH
    if rem:
        pltpu.sync_copy(
            rows[0].at[pl.ds(0, rem)],
            acc.at[pl.ds(sid * _RPA + (_RPA // _ACH) * _ACH, rem)],
        )
    plsc.subcore_barrier()

    half_n = _ACPT // 4
    for half in range(4):
        base = tid * _ACPT + half * half_n
        pltpu.sync_copy(src_hbm.at[pl.ds(base, half_n)], srcv)
        pltpu.sync_copy(dst_hbm.at[pl.ds(base, half_n)], dstv)

        pltpu.async_copy(y_hbm.at[srcv.at[0]], rows[0], sems[0])
        pltpu.async_copy(y_hbm.at[srcv.at[1]], rows[1], sems[1])

        def step(k, carry):
            g = k * _NBUF
            for b in range(_NBUF):
                c = g + b
                b2 = (b + 2) % _NBUF

                @pl.when(c >= 2)
                def _():
                    pltpu.make_async_copy(rows[b2], acc.at[dstv.at[c - 2]],
                                          ssems[b2]).wait()

                @pl.when(c + 2 < half_n)
                def _():
                    pltpu.async_copy(y_hbm.at[srcv.at[c + 2]], rows[b2],
                                     sems[b2])

                pltpu.make_async_copy(y_hbm.at[srcv.at[c]], rows[b],
                                      sems[b]).wait()
                pltpu.async_copy(rows[b], acc.at[dstv.at[c]], ssems[b],
                                 add=True)

            return carry

        lax.fori_loop(0, half_n // _NBUF, step, 0)

        pltpu.make_async_copy(rows[(half_n - 2) % _NBUF],
                              acc.at[dstv.at[half_n - 2]],
                              ssems[(half_n - 2) % _NBUF]).wait()
        pltpu.make_async_copy(rows[(half_n - 1) % _NBUF],
                              acc.at[dstv.at[half_n - 1]],
                              ssems[(half_n - 1) % _NBUF]).wait()

    plsc.subcore_barrier()
    pltpu.sync_copy(
        acc.at[pl.ds(sid * _RPA, _RPA)],
        out_hbm.at[cid, pl.ds(sid * _RPA, _RPA)],
    )


@functools.cache
def _make_agg(f):
    return functools.partial(
        pl.kernel,
        out_type=jax.ShapeDtypeStruct((_NC, _ACC_A, f), jnp.float32),
        mesh=_mesh(),
        scratch_types=[
            pltpu.VMEM((_ACPT // 4, _ACH), jnp.int32),
            pltpu.VMEM((_ACPT // 4, _ACH), jnp.int32),
            [pltpu.VMEM((_ACH, f), jnp.float32) for _ in range(_NBUF)],
            pltpu.VMEM_SHARED((_ACC_A, f), jnp.float32),
            [pltpu.SemaphoreType.DMA for _ in range(_NBUF)],
            [pltpu.SemaphoreType.DMA for _ in range(_NBUF)],
        ],
    )(functools.partial(_agg_body, f))

_BLK = 1000
_GRID = _N // _BLK


def _dense1_body(x_ref, w_ref, deg_ref, y_ref):
    dinv = lax.rsqrt(deg_ref[0] + deg_ref[1] + 1.0)
    y_ref[...] = dinv * jnp.dot(
        x_ref[...], w_ref[...], preferred_element_type=jnp.float32
    )


def _dense2_body(agg_ref, y1_ref, deg_ref, b1_ref, q_ref):
    dinv = lax.rsqrt(deg_ref[0] + deg_ref[1] + 1.0)
    s = agg_ref[0] + agg_ref[1] + y1_ref[...]
    q_ref[...] = dinv * jnp.maximum(dinv * s + b1_ref[...], 0.0)


def _dense3_body(agg_ref, q_ref, deg_ref, w2_ref, b2_ref, out_ref):
    dinv = lax.rsqrt(deg_ref[0] + deg_ref[1] + 1.0)
    s = agg_ref[0] + agg_ref[1] + q_ref[...]
    out_ref[...] = dinv * jnp.dot(
        s, w2_ref[...], preferred_element_type=jnp.float32
    ) + b2_ref[...]


def _row_spec(f):
    return pl.BlockSpec((_BLK, f), lambda i: (i, 0))


def _full_spec(shape):
    return pl.BlockSpec(shape, lambda i: tuple(0 for _ in shape))


_deg_spec = pl.BlockSpec((_NC, _BLK, 1), lambda i: (0, i, 0))
_agg_spec_h = pl.BlockSpec((_NC, _BLK, _NHID), lambda i: (0, i, 0))

_dense1 = pl.pallas_call(
    _dense1_body,
    grid=(_GRID,),
    in_specs=[_row_spec(_NFEAT), _full_spec((_NFEAT, _NHID)), _deg_spec],
    out_specs=_row_spec(_NHID),
    out_shape=jax.ShapeDtypeStruct((_N, _NHID), jnp.float32),
)

_dense2 = pl.pallas_call(
    _dense2_body,
    grid=(_GRID,),
    in_specs=[
        _agg_spec_h,
        _row_spec(_NHID),
        _deg_spec,
        _full_spec((1, _NHID)),
    ],
    out_specs=_row_spec(_NHID),
    out_shape=jax.ShapeDtypeStruct((_N, _NHID), jnp.float32),
)

_dense3 = pl.pallas_call(
    _dense3_body,
    grid=(_GRID,),
    in_specs=[
        _agg_spec_h,
        _row_spec(_NHID),
        _deg_spec,
        _full_spec((_NHID, _NCLASS)),
        _full_spec((1, _NCLASS)),
    ],
    out_specs=_row_spec(_NCLASS),
    out_shape=jax.ShapeDtypeStruct((_N, _NCLASS), jnp.float32),
)


def kernel(x, edge_index, W1, b1, W2, b2):
    src = edge_index[0]
    dst = edge_index[1]
    pad = _E_PAD - _E
    pad_idx = jnp.arange(pad, dtype=jnp.int32)
    src_f = jnp.concatenate([src, pad_idx % _N])
    dst_f = jnp.concatenate([dst, _N + (pad_idx % 16)])
    dst2d = dst_f.reshape(_CHUNKS_PAD, _CH)
    src_a = src_f.reshape(_NT * _ACPT, _ACH)
    dst_a = dst_f.reshape(_NT * _ACPT, _ACH)

    deg_p = _deg_call()(dst2d).reshape(_NC, _ACC_ROWS)
    deg_col = deg_p[:, :, None]

    y1 = _dense1(x, W1, deg_col)
    agg1 = _make_agg(_NHID)(y1, src_a, dst_a)
    q = _dense2(agg1, y1, deg_col, b1.reshape(1, _NHID))
    agg2 = _make_agg(_NHID)(q, src_a, dst_a)
    return _dense3(agg2, q, deg_col, W2, b2.reshape(1, _NCLASS))


# speedup vs baseline: 1.0604x; 1.0604x over previous
import functools

import jax
import jax.numpy as jnp
from jax import lax
from jax.experimental import pallas as pl
from jax.experimental.pallas import tpu as pltpu
from jax.experimental.pallas import tpu_sc as plsc

_N = 10000
_E = 320000
_NFEAT = 128
_NHID = 128
_NCLASS = 64

_NC = 2
_NS = 16
_NT = _NC * _NS
_CH = 128
_CPT = 80
_CHUNKS_PAD = _CPT * _NT
_E_PAD = _CHUNKS_PAD * _CH
_ACH = 64
_ACPT = _E_PAD // (_NT * _ACH)
_NBUF = 4
_ACC_ROWS = 10240
_RPT = _ACC_ROWS // _NS
_ACC_A = 10112
_RPA = _ACC_A // _NS
_HALF = _CPT // 2

@functools.cache
def _mesh():
    return plsc.VectorSubcoreMesh(
        core_axis_name="c", subcore_axis_name="s", num_cores=_NC, num_subcores=_NS
    )


def _deg_body(dst_hbm, out_hbm, idx_v, ones_v, zero_v, acc):
    cid = lax.axis_index("c")
    sid = lax.axis_index("s")
    tid = cid * _NS + sid
    for i in range(_CH // 16):
        ones_v[pl.ds(i * 16, 16)] = jnp.ones((16,), jnp.float32)
    for i in range(_RPT // 16):
        zero_v[pl.ds(i * 16, 16)] = jnp.zeros((16,), jnp.float32)
    pltpu.sync_copy(zero_v, acc.at[pl.ds(sid * _RPT, _RPT)])
    plsc.subcore_barrier()
    pltpu.sync_copy(dst_hbm.at[pl.ds(tid * _CPT, _CPT)], idx_v)

    def body(i, carry):
        pltpu.sync_copy(ones_v, acc.at[idx_v.at[i]], add=True)
        return carry

    lax.fori_loop(0, _CPT, body, 0)
    plsc.subcore_barrier()
    pltpu.sync_copy(
        acc.at[pl.ds(sid * _RPT, _RPT)],
        out_hbm.at[pl.ds(cid * _ACC_ROWS + sid * _RPT, _RPT)],
    )


@functools.cache
def _deg_call():
    return functools.partial(
        pl.kernel,
        out_type=jax.ShapeDtypeStruct((_NC * _ACC_ROWS,), jnp.float32),
        mesh=_mesh(),
        scratch_types=[
            pltpu.VMEM((_CPT, _CH), jnp.int32),
            pltpu.VMEM((_CH,), jnp.float32),
            pltpu.VMEM((_RPT,), jnp.float32),
            pltpu.VMEM_SHARED((_ACC_ROWS,), jnp.float32),
        ],
    )(_deg_body)


def _agg_body(f, y_hbm, src_hbm, dst_hbm, out_hbm, srcv, dstv, rows, acc, sems):
    cid = lax.axis_index("c")
    sid = lax.axis_index("s")
    tid = cid * _NS + sid

    def zfill(r, carry):
        for c in range(f // 16):
            rows[0][r, pl.ds(c * 16, 16)] = jnp.zeros((16,), jnp.float32)
        return carry

    lax.fori_loop(0, _ACH, zfill, 0)
    for k in range(_RPA // _ACH):
        pltpu.sync_copy(rows[0], acc.at[pl.ds(sid * _RPA + k * _ACH, _ACH)])
    rem = _RPA % _ACH
    if rem:
        pltpu.sync_copy(
            rows[0].at[pl.ds(0, rem)],
            acc.at[pl.ds(sid * _RPA + (_RPA // _ACH) * _ACH, rem)],
        )
    plsc.subcore_barrier()

    half_n = _ACPT // 4
    for half in range(4):
        base = tid * _ACPT + half * half_n
        pltpu.sync_copy(src_hbm.at[pl.ds(base, half_n)], srcv)
        pltpu.sync_copy(dst_hbm.at[pl.ds(base, half_n)], dstv)

        for b in range(_NBUF):
            pltpu.async_copy(y_hbm.at[srcv.at[b]], rows[b], sems[b])

        def step(k, carry):
            g = k * _NBUF
            for b in range(_NBUF):
                c = g + b
                pltpu.make_async_copy(y_hbm.at[srcv.at[c]], rows[b],
                                      sems[b]).wait()
                pltpu.sync_copy(rows[b], acc.at[dstv.at[c]], add=True)

                @pl.when(c + _NBUF < half_n)
                def _():
                    pltpu.async_copy(y_hbm.at[srcv.at[c + _NBUF]], rows[b],
                                     sems[b])

            return carry

        lax.fori_loop(0, half_n // _NBUF, step, 0)

    plsc.subcore_barrier()
    pltpu.sync_copy(
        acc.at[pl.ds(sid * _RPA, _RPA)],
        out_hbm.at[cid, pl.ds(sid * _RPA, _RPA)],
    )


@functools.cache
def _make_agg(f):
    return functools.partial(
        pl.kernel,
        out_type=jax.ShapeDtypeStruct((_NC, _ACC_A, f), jnp.float32),
        mesh=_mesh(),
        scratch_types=[
            pltpu.VMEM((_ACPT // 4, _ACH), jnp.int32),
            pltpu.VMEM((_ACPT // 4, _ACH), jnp.int32),
            [pltpu.VMEM((_ACH, f), jnp.float32) for _ in range(_NBUF)],
            pltpu.VMEM_SHARED((_ACC_A, f), jnp.float32),
            [pltpu.SemaphoreType.DMA for _ in range(_NBUF)],
        ],
    )(functools.partial(_agg_body, f))

_BLK = 1000
_GRID = _N // _BLK


def _dense1_body(x_ref, w_ref, deg_ref, y_ref):
    dinv = lax.rsqrt(deg_ref[0] + deg_ref[1] + 1.0)
    y_ref[...] = dinv * jnp.dot(
        x_ref[...], w_ref[...], preferred_element_type=jnp.float32
    )


def _dense2_body(agg_ref, y1_ref, deg_ref, b1_ref, q_ref):
    dinv = lax.rsqrt(deg_ref[0] + deg_ref[1] + 1.0)
    s = agg_ref[0] + agg_ref[1] + y1_ref[...]
    q_ref[...] = dinv * jnp.maximum(dinv * s + b1_ref[...], 0.0)


def _dense3_body(agg_ref, q_ref, deg_ref, w2_ref, b2_ref, out_ref):
    dinv = lax.rsqrt(deg_ref[0] + deg_ref[1] + 1.0)
    s = agg_ref[0] + agg_ref[1] + q_ref[...]
    out_ref[...] = dinv * jnp.dot(
        s, w2_ref[...], preferred_element_type=jnp.float32
    ) + b2_ref[...]


def _row_spec(f):
    return pl.BlockSpec((_BLK, f), lambda i: (i, 0))


def _full_spec(shape):
    return pl.BlockSpec(shape, lambda i: tuple(0 for _ in shape))


_deg_spec = pl.BlockSpec((_NC, _BLK, 1), lambda i: (0, i, 0))
_agg_spec_h = pl.BlockSpec((_NC, _BLK, _NHID), lambda i: (0, i, 0))

_dense1 = pl.pallas_call(
    _dense1_body,
    grid=(_GRID,),
    in_specs=[_row_spec(_NFEAT), _full_spec((_NFEAT, _NHID)), _deg_spec],
    out_specs=_row_spec(_NHID),
    out_shape=jax.ShapeDtypeStruct((_N, _NHID), jnp.float32),
)

_dense2 = pl.pallas_call(
    _dense2_body,
    grid=(_GRID,),
    in_specs=[
        _agg_spec_h,
        _row_spec(_NHID),
        _deg_spec,
        _full_spec((1, _NHID)),
    ],
    out_specs=_row_spec(_NHID),
    out_shape=jax.ShapeDtypeStruct((_N, _NHID), jnp.float32),
)

_dense3 = pl.pallas_call(
    _dense3_body,
    grid=(_GRID,),
    in_specs=[
        _agg_spec_h,
        _row_spec(_NHID),
        _deg_spec,
        _full_spec((_NHID, _NCLASS)),
        _full_spec((1, _NCLASS)),
    ],
    out_specs=_row_spec(_NCLASS),
    out_shape=jax.ShapeDtypeStruct((_N, _NCLASS), jnp.float32),
)


def kernel(x, edge_index, W1, b1, W2, b2):
    src = edge_index[0]
    dst = edge_index[1]
    pad = _E_PAD - _E
    pad_idx = jnp.arange(pad, dtype=jnp.int32)
    src_f = jnp.concatenate([src, pad_idx % _N])
    dst_f = jnp.concatenate([dst, _N + (pad_idx % 16)])
    dst2d = dst_f.reshape(_CHUNKS_PAD, _CH)
    src_a = src_f.reshape(_NT * _ACPT, _ACH)
    dst_a = dst_f.reshape(_NT * _ACPT, _ACH)

    deg_p = _deg_call()(dst2d).reshape(_NC, _ACC_ROWS)
    deg_col = deg_p[:, :, None]

    y1 = _dense1(x, W1, deg_col)
    agg1 = _make_agg(_NHID)(y1, src_a, dst_a)
    q = _dense2(agg1, y1, deg_col, b1.reshape(1, _NHID))
    agg2 = _make_agg(_NHID)(q, src_a, dst_a)
    return _dense3(agg2, q, deg_col, W2, b2.reshape(1, _NCLASS))


# speedup vs baseline: 1.0769x; 1.0156x over previous
import functools

import jax
import jax.numpy as jnp
from jax import lax
from jax.experimental import pallas as pl
from jax.experimental.pallas import tpu as pltpu
from jax.experimental.pallas import tpu_sc as plsc

_N = 10000
_E = 320000
_NFEAT = 128
_NHID = 128
_NCLASS = 64

_NC = 2
_NS = 16
_NT = _NC * _NS
_CH = 128
_CPT = 80
_CHUNKS_PAD = _CPT * _NT
_E_PAD = _CHUNKS_PAD * _CH
_ACH = 64
_ACPT = _E_PAD // (_NT * _ACH)
_NBUF = 4
_ACC_ROWS = 10240
_RPT = _ACC_ROWS // _NS
_ACC_A = 10112
_RPA = _ACC_A // _NS
_HALF = _CPT // 2

@functools.cache
def _mesh():
    return plsc.VectorSubcoreMesh(
        core_axis_name="c", subcore_axis_name="s", num_cores=_NC, num_subcores=_NS
    )


def _deg_body(dst_hbm, out_hbm, idx_v, ones_v, zero_v, acc, dsem):
    cid = lax.axis_index("c")
    sid = lax.axis_index("s")
    tid = cid * _NS + sid
    for i in range(_CH // 16):
        ones_v[pl.ds(i * 16, 16)] = jnp.ones((16,), jnp.float32)
    for i in range(_RPT // 16):
        zero_v[pl.ds(i * 16, 16)] = jnp.zeros((16,), jnp.float32)
    pltpu.sync_copy(zero_v, acc.at[pl.ds(sid * _RPT, _RPT)])
    plsc.subcore_barrier()
    pltpu.sync_copy(dst_hbm.at[pl.ds(tid * _CPT, _CPT)], idx_v)

    _DEPTH = 8

    def prime(i, carry):
        pltpu.async_copy(ones_v, acc.at[idx_v.at[i]], dsem, add=True)
        return carry

    lax.fori_loop(0, _DEPTH, prime, 0)

    def body(i, carry):
        pltpu.make_async_copy(ones_v, acc.at[idx_v.at[0]], dsem).wait()
        pltpu.async_copy(ones_v, acc.at[idx_v.at[i + _DEPTH]], dsem, add=True)
        return carry

    lax.fori_loop(0, _CPT - _DEPTH, body, 0)

    def drain(i, carry):
        pltpu.make_async_copy(ones_v, acc.at[idx_v.at[0]], dsem).wait()
        return carry

    lax.fori_loop(0, _DEPTH, drain, 0)
    plsc.subcore_barrier()
    pltpu.sync_copy(
        acc.at[pl.ds(sid * _RPT, _RPT)],
        out_hbm.at[pl.ds(cid * _ACC_ROWS + sid * _RPT, _RPT)],
    )


@functools.cache
def _deg_call():
    return functools.partial(
        pl.kernel,
        out_type=jax.ShapeDtypeStruct((_NC * _ACC_ROWS,), jnp.float32),
        mesh=_mesh(),
        scratch_types=[
            pltpu.VMEM((_CPT, _CH), jnp.int32),
            pltpu.VMEM((_CH,), jnp.float32),
            pltpu.VMEM((_RPT,), jnp.float32),
            pltpu.VMEM_SHARED((_ACC_ROWS,), jnp.float32),
            pltpu.SemaphoreType.DMA,
        ],
    )(_deg_body)


def _agg_body(f, y_hbm, src_hbm, dst_hbm, out_hbm, srcv, dstv, rows, acc, sems):
    cid = lax.axis_index("c")
    sid = lax.axis_index("s")
    tid = cid * _NS + sid

    def zfill(r, carry):
        for c in range(f // 16):
            rows[0][r, pl.ds(c * 16, 16)] = jnp.zeros((16,), jnp.float32)
        return carry

    lax.fori_loop(0, _ACH, zfill, 0)
    for k in range(_RPA // _ACH):
        pltpu.sync_copy(rows[0], acc.at[pl.ds(sid * _RPA + k * _ACH, _ACH)])
    rem = _RPA % _ACH
    if rem:
        pltpu.sync_copy(
            rows[0].at[pl.ds(0, rem)],
            acc.at[pl.ds(sid * _RPA + (_RPA // _ACH) * _ACH, rem)],
        )
    plsc.subcore_barrier()

    half_n = _ACPT // 4

    def stage_body(half, carry0):
        base = pl.multiple_of(tid * _ACPT + half * half_n, 8)
        pltpu.sync_copy(src_hbm.at[pl.ds(base, half_n)], srcv)
        pltpu.sync_copy(dst_hbm.at[pl.ds(base, half_n)], dstv)

        for b in range(_NBUF):
            pltpu.async_copy(y_hbm.at[srcv.at[b]], rows[b], sems[b])

        def step(k, carry):
            g = k * _NBUF
            for b in range(_NBUF):
                c = g + b
                pltpu.make_async_copy(y_hbm.at[srcv.at[c]], rows[b],
                                      sems[b]).wait()
                pltpu.sync_copy(rows[b], acc.at[dstv.at[c]], add=True)

                @pl.when(c + _NBUF < half_n)
                def _():
                    pltpu.async_copy(y_hbm.at[srcv.at[c + _NBUF]], rows[b],
                                     sems[b])

            return carry

        lax.fori_loop(0, half_n // _NBUF, step, 0)
        return carry0

    lax.fori_loop(0, 4, stage_body, 0)

    plsc.subcore_barrier()
    pltpu.sync_copy(
        acc.at[pl.ds(sid * _RPA, _RPA)],
        out_hbm.at[cid, pl.ds(sid * _RPA, _RPA)],
    )


@functools.cache
def _make_agg(f):
    return functools.partial(
        pl.kernel,
        out_type=jax.ShapeDtypeStruct((_NC, _ACC_A, f), jnp.float32),
        mesh=_mesh(),
        scratch_types=[
            pltpu.VMEM((_ACPT // 4, _ACH), jnp.int32),
            pltpu.VMEM((_ACPT // 4, _ACH), jnp.int32),
            [pltpu.VMEM((_ACH, f), jnp.float32) for _ in range(_NBUF)],
            pltpu.VMEM_SHARED((_ACC_A, f), jnp.float32),
            [pltpu.SemaphoreType.DMA for _ in range(_NBUF)],
        ],
    )(functools.partial(_agg_body, f))

_BLK = 1000
_GRID = _N // _BLK


def _dense1_body(x_ref, w_ref, deg_ref, y_ref):
    dinv = lax.rsqrt(deg_ref[0] + deg_ref[1] + 1.0)
    y_ref[...] = dinv * jnp.dot(
        x_ref[...], w_ref[...], preferred_element_type=jnp.float32
    )


def _dense2_body(agg_ref, y1_ref, deg_ref, b1_ref, q_ref):
    dinv = lax.rsqrt(deg_ref[0] + deg_ref[1] + 1.0)
    s = agg_ref[0] + agg_ref[1] + y1_ref[...]
    q_ref[...] = dinv * jnp.maximum(dinv * s + b1_ref[...], 0.0)


def _dense3_body(agg_ref, q_ref, deg_ref, w2_ref, b2_ref, out_ref):
    dinv = lax.rsqrt(deg_ref[0] + deg_ref[1] + 1.0)
    s = agg_ref[0] + agg_ref[1] + q_ref[...]
    out_ref[...] = dinv * jnp.dot(
        s, w2_ref[...], preferred_element_type=jnp.float32
    ) + b2_ref[...]


def _row_spec(f):
    return pl.BlockSpec((_BLK, f), lambda i: (i, 0))


def _full_spec(shape):
    return pl.BlockSpec(shape, lambda i: tuple(0 for _ in shape))


_deg_spec = pl.BlockSpec((_NC, _BLK, 1), lambda i: (0, i, 0))
_agg_spec_h = pl.BlockSpec((_NC, _BLK, _NHID), lambda i: (0, i, 0))

_dense1 = pl.pallas_call(
    _dense1_body,
    grid=(_GRID,),
    in_specs=[_row_spec(_NFEAT), _full_spec((_NFEAT, _NHID)), _deg_spec],
    out_specs=_row_spec(_NHID),
    out_shape=jax.ShapeDtypeStruct((_N, _NHID), jnp.float32),
)

_dense2 = pl.pallas_call(
    _dense2_body,
    grid=(_GRID,),
    in_specs=[
        _agg_spec_h,
        _row_spec(_NHID),
        _deg_spec,
        _full_spec((1, _NHID)),
    ],
    out_specs=_row_spec(_NHID),
    out_shape=jax.ShapeDtypeStruct((_N, _NHID), jnp.float32),
)

_dense3 = pl.pallas_call(
    _dense3_body,
    grid=(_GRID,),
    in_specs=[
        _agg_spec_h,
        _row_spec(_NHID),
        _deg_spec,
        _full_spec((_NHID, _NCLASS)),
        _full_spec((1, _NCLASS)),
    ],
    out_specs=_row_spec(_NCLASS),
    out_shape=jax.ShapeDtypeStruct((_N, _NCLASS), jnp.float32),
)


def kernel(x, edge_index, W1, b1, W2, b2):
    src = edge_index[0]
    dst = edge_index[1]
    pad = _E_PAD - _E
    pad_idx = jnp.arange(pad, dtype=jnp.int32)
    src_f = jnp.concatenate([src, pad_idx % _N])
    dst_f = jnp.concatenate([dst, _N + (pad_idx % 16)])
    dst2d = dst_f.reshape(_CHUNKS_PAD, _CH)
    src_a = src_f.reshape(_NT * _ACPT, _ACH)
    dst_a = dst_f.reshape(_NT * _ACPT, _ACH)

    deg_p = _deg_call()(dst2d).reshape(_NC, _ACC_ROWS)
    deg_col = deg_p[:, :, None]

    y1 = _dense1(x, W1, deg_col)
    agg1 = _make_agg(_NHID)(y1, src_a, dst_a)
    q = _dense2(agg1, y1, deg_col, b1.reshape(1, _NHID))
    agg2 = _make_agg(_NHID)(q, src_a, dst_a)
    return _dense3(agg2, q, deg_col, W2, b2.reshape(1, _NCLASS))


# speedup vs baseline: 1.1010x; 1.0224x over previous
import functools

import jax
import jax.numpy as jnp
from jax import lax
from jax.experimental import pallas as pl
from jax.experimental.pallas import tpu as pltpu
from jax.experimental.pallas import tpu_sc as plsc

_N = 10000
_E = 320000
_NFEAT = 128
_NHID = 128
_NCLASS = 64

_NC = 2
_NS = 16
_NT = _NC * _NS
_CH = 128
_CPT = 80
_CHUNKS_PAD = _CPT * _NT
_E_PAD = _CHUNKS_PAD * _CH
_ACH = 64
_ACPT = _E_PAD // (_NT * _ACH)
_NBUF = 4
_ACC_ROWS = 10240
_RPT = _ACC_ROWS // _NS
_ACC_A = 10112
_RPA = _ACC_A // _NS
_HALF = _CPT // 2

@functools.cache
def _mesh():
    return plsc.VectorSubcoreMesh(
        core_axis_name="c", subcore_axis_name="s", num_cores=_NC, num_subcores=_NS
    )


def _deg_body(dst_hbm, out_hbm, idx_v, ones_v, zero_v, acc, dsem):
    cid = lax.axis_index("c")
    sid = lax.axis_index("s")
    tid = cid * _NS + sid
    for i in range(_CH // 16):
        ones_v[pl.ds(i * 16, 16)] = jnp.ones((16,), jnp.float32)
    for i in range(_RPT // 16):
        zero_v[pl.ds(i * 16, 16)] = jnp.zeros((16,), jnp.float32)
    pltpu.sync_copy(zero_v, acc.at[pl.ds(sid * _RPT, _RPT)])
    plsc.subcore_barrier()
    pltpu.sync_copy(dst_hbm.at[pl.ds(tid * _CPT, _CPT)], idx_v)

    _DEPTH = 8

    def prime(i, carry):
        pltpu.async_copy(ones_v, acc.at[idx_v.at[i]], dsem, add=True)
        return carry

    lax.fori_loop(0, _DEPTH, prime, 0)

    def body(i, carry):
        pltpu.make_async_copy(ones_v, acc.at[idx_v.at[0]], dsem).wait()
        pltpu.async_copy(ones_v, acc.at[idx_v.at[i + _DEPTH]], dsem, add=True)
        return carry

    lax.fori_loop(0, _CPT - _DEPTH, body, 0)

    def drain(i, carry):
        pltpu.make_async_copy(ones_v, acc.at[idx_v.at[0]], dsem).wait()
        return carry

    lax.fori_loop(0, _DEPTH, drain, 0)
    plsc.subcore_barrier()
    pltpu.sync_copy(
        acc.at[pl.ds(sid * _RPT, _RPT)],
        out_hbm.at[pl.ds(cid * _ACC_ROWS + sid * _RPT, _RPT)],
    )


@functools.cache
def _deg_call():
    return functools.partial(
        pl.kernel,
        out_type=jax.ShapeDtypeStruct((_NC * _ACC_ROWS,), jnp.float32),
        mesh=_mesh(),
        scratch_types=[
            pltpu.VMEM((_CPT, _CH), jnp.int32),
            pltpu.VMEM((_CH,), jnp.float32),
            pltpu.VMEM((_RPT,), jnp.float32),
            pltpu.VMEM_SHARED((_ACC_ROWS,), jnp.float32),
            pltpu.SemaphoreType.DMA,
        ],
    )(_deg_body)


def _agg_body(f, y_hbm, src_hbm, dst_hbm, out_hbm, srcv, dstv, rows, acc, sems):
    cid = lax.axis_index("c")
    sid = lax.axis_index("s")
    tid = cid * _NS + sid

    def zfill(r, carry):
        for c in range(f // 16):
            rows[0][r, pl.ds(c * 16, 16)] = jnp.zeros((16,), jnp.float32)
        return carry

    lax.fori_loop(0, _ACH, zfill, 0)
    for k in range(_RPA // _ACH):
        pltpu.async_copy(rows[0], acc.at[pl.ds(sid * _RPA + k * _ACH, _ACH)],
                         sems[k % _NBUF])
    rem = _RPA % _ACH
    if rem:
        pltpu.async_copy(
            rows[0].at[pl.ds(0, rem)],
            acc.at[pl.ds(sid * _RPA + (_RPA // _ACH) * _ACH, rem)],
            sems[(_RPA // _ACH) % _NBUF],
        )
    for k in range(_RPA // _ACH):
        pltpu.make_async_copy(
            rows[0], acc.at[pl.ds(sid * _RPA + k * _ACH, _ACH)],
            sems[k % _NBUF]).wait()
    if rem:
        pltpu.make_async_copy(
            rows[0].at[pl.ds(0, rem)],
            acc.at[pl.ds(sid * _RPA + (_RPA // _ACH) * _ACH, rem)],
            sems[(_RPA // _ACH) % _NBUF]).wait()
    plsc.subcore_barrier()

    half_n = _ACPT // 4

    def stage_body(half, carry0):
        base = pl.multiple_of(tid * _ACPT + half * half_n, 8)
        pltpu.async_copy(src_hbm.at[pl.ds(base, half_n)], srcv, sems[0])
        pltpu.async_copy(dst_hbm.at[pl.ds(base, half_n)], dstv, sems[1])
        pltpu.make_async_copy(src_hbm.at[pl.ds(base, half_n)], srcv,
                              sems[0]).wait()
        pltpu.make_async_copy(dst_hbm.at[pl.ds(base, half_n)], dstv,
                              sems[1]).wait()

        for b in range(_NBUF):
            pltpu.async_copy(y_hbm.at[srcv.at[b]], rows[b], sems[b])

        def step(k, carry):
            g = k * _NBUF
            for b in range(_NBUF):
                c = g + b
                pltpu.make_async_copy(y_hbm.at[srcv.at[c]], rows[b],
                                      sems[b]).wait()
                pltpu.sync_copy(rows[b], acc.at[dstv.at[c]], add=True)

                @pl.when(c + _NBUF < half_n)
                def _():
                    pltpu.async_copy(y_hbm.at[srcv.at[c + _NBUF]], rows[b],
                                     sems[b])

            return carry

        lax.fori_loop(0, half_n // _NBUF, step, 0)
        return carry0

    lax.fori_loop(0, 4, stage_body, 0)

    plsc.subcore_barrier()
    pltpu.sync_copy(
        acc.at[pl.ds(sid * _RPA, _RPA)],
        out_hbm.at[cid, pl.ds(sid * _RPA, _RPA)],
    )


@functools.cache
def _make_agg(f):
    return functools.partial(
        pl.kernel,
        out_type=jax.ShapeDtypeStruct((_NC, _ACC_A, f), jnp.float32),
        mesh=_mesh(),
        scratch_types=[
            pltpu.VMEM((_ACPT // 4, _ACH), jnp.int32),
            pltpu.VMEM((_ACPT // 4, _ACH), jnp.int32),
            [pltpu.VMEM((_ACH, f), jnp.float32) for _ in range(_NBUF)],
            pltpu.VMEM_SHARED((_ACC_A, f), jnp.float32),
            [pltpu.SemaphoreType.DMA for _ in range(_NBUF)],
        ],
    )(functools.partial(_agg_body, f))

_BLK = 1000
_GRID = _N // _BLK


def _dense1_body(x_ref, w_ref, deg_ref, y_ref):
    dinv = lax.rsqrt(deg_ref[0] + deg_ref[1] + 1.0)
    y_ref[...] = dinv * jnp.dot(
        x_ref[...], w_ref[...], preferred_element_type=jnp.float32
    )


def _dense2_body(agg_ref, y1_ref, deg_ref, b1_ref, q_ref):
    dinv = lax.rsqrt(deg_ref[0] + deg_ref[1] + 1.0)
    s = agg_ref[0] + agg_ref[1] + y1_ref[...]
    q_ref[...] = dinv * jnp.maximum(dinv * s + b1_ref[...], 0.0)


def _dense3_body(agg_ref, q_ref, deg_ref, w2_ref, b2_ref, out_ref):
    dinv = lax.rsqrt(deg_ref[0] + deg_ref[1] + 1.0)
    s = agg_ref[0] + agg_ref[1] + q_ref[...]
    out_ref[...] = dinv * jnp.dot(
        s, w2_ref[...], preferred_element_type=jnp.float32
    ) + b2_ref[...]


def _row_spec(f):
    return pl.BlockSpec((_BLK, f), lambda i: (i, 0))


def _full_spec(shape):
    return pl.BlockSpec(shape, lambda i: tuple(0 for _ in shape))


_deg_spec = pl.BlockSpec((_NC, _BLK, 1), lambda i: (0, i, 0))
_agg_spec_h = pl.BlockSpec((_NC, _BLK, _NHID), lambda i: (0, i, 0))

_dense1 = pl.pallas_call(
    _dense1_body,
    grid=(_GRID,),
    in_specs=[_row_spec(_NFEAT), _full_spec((_NFEAT, _NHID)), _deg_spec],
    out_specs=_row_spec(_NHID),
    out_shape=jax.ShapeDtypeStruct((_N, _NHID), jnp.float32),
)

_dense2 = pl.pallas_call(
    _dense2_body,
    grid=(_GRID,),
    in_specs=[
        _agg_spec_h,
        _row_spec(_NHID),
        _deg_spec,
        _full_spec((1, _NHID)),
    ],
    out_specs=_row_spec(_NHID),
    out_shape=jax.ShapeDtypeStruct((_N, _NHID), jnp.float32),
)

_dense3 = pl.pallas_call(
    _dense3_body,
    grid=(_GRID,),
    in_specs=[
        _agg_spec_h,
        _row_spec(_NHID),
        _deg_spec,
        _full_spec((_NHID, _NCLASS)),
        _full_spec((1, _NCLASS)),
    ],
    out_specs=_row_spec(_NCLASS),
    out_shape=jax.ShapeDtypeStruct((_N, _NCLASS), jnp.float32),
)


def kernel(x, edge_index, W1, b1, W2, b2):
    src = edge_index[0]
    dst = edge_index[1]
    pad = _E_PAD - _E
    pad_idx = jnp.arange(pad, dtype=jnp.int32)
    src_f = jnp.concatenate([src, pad_idx % _N])
    dst_f = jnp.concatenate([dst, _N + (pad_idx % 16)])
    dst2d = dst_f.reshape(_CHUNKS_PAD, _CH)
    src_a = src_f.reshape(_NT * _ACPT, _ACH)
    dst_a = dst_f.reshape(_NT * _ACPT, _ACH)

    deg_p = _deg_call()(dst2d).reshape(_NC, _ACC_ROWS)
    deg_col = deg_p[:, :, None]

    y1 = _dense1(x, W1, deg_col)
    agg1 = _make_agg(_NHID)(y1, src_a, dst_a)
    q = _dense2(agg1, y1, deg_col, b1.reshape(1, _NHID))
    agg2 = _make_agg(_NHID)(q, src_a, dst_a)
    return _dense3(agg2, q, deg_col, W2, b2.reshape(1, _NCLASS))


# speedup vs baseline: 1.1017x; 1.0007x over previous
import functools

import jax
import jax.numpy as jnp
from jax import lax
from jax.experimental import pallas as pl
from jax.experimental.pallas import tpu as pltpu
from jax.experimental.pallas import tpu_sc as plsc

_N = 10000
_E = 320000
_NFEAT = 128
_NHID = 128
_NCLASS = 64

_NC = 2
_NS = 16
_NT = _NC * _NS
_CH = 128
_CPT = 80
_CHUNKS_PAD = _CPT * _NT
_E_PAD = _CHUNKS_PAD * _CH
_ACH = 64
_ACPT = _E_PAD // (_NT * _ACH)
_NBUF = 4
_ACC_ROWS = 10240
_RPT = _ACC_ROWS // _NS
_ACC_A = 10112
_RPA = _ACC_A // _NS

@functools.cache
def _mesh():
    return plsc.VectorSubcoreMesh(
        core_axis_name="c", subcore_axis_name="s", num_cores=_NC, num_subcores=_NS
    )


def _deg_body(dst_hbm, out_hbm, idx_v, ones_v, zero_v, acc, dsem):
    cid = lax.axis_index("c")
    sid = lax.axis_index("s")
    tid = cid * _NS + sid
    for i in range(_CH // 16):
        ones_v[pl.ds(i * 16, 16)] = jnp.ones((16,), jnp.float32)
    for i in range(_RPT // 16):
        zero_v[pl.ds(i * 16, 16)] = jnp.zeros((16,), jnp.float32)
    pltpu.sync_copy(zero_v, acc.at[pl.ds(sid * _RPT, _RPT)])
    plsc.subcore_barrier()
    pltpu.sync_copy(dst_hbm.at[pl.ds(tid * _CPT, _CPT)], idx_v)

    _DEPTH = 8

    def prime(i, carry):
        pltpu.async_copy(ones_v, acc.at[idx_v.at[i]], dsem, add=True)
        return carry

    lax.fori_loop(0, _DEPTH, prime, 0)

    def body(i, carry):
        pltpu.make_async_copy(ones_v, acc.at[idx_v.at[0]], dsem).wait()
        pltpu.async_copy(ones_v, acc.at[idx_v.at[i + _DEPTH]], dsem, add=True)
        return carry

    lax.fori_loop(0, _CPT - _DEPTH, body, 0)

    def drain(i, carry):
        pltpu.make_async_copy(ones_v, acc.at[idx_v.at[0]], dsem).wait()
        return carry

    lax.fori_loop(0, _DEPTH, drain, 0)
    plsc.subcore_barrier()
    pltpu.sync_copy(
        acc.at[pl.ds(sid * _RPT, _RPT)],
        out_hbm.at[pl.ds(cid * _ACC_ROWS + sid * _RPT, _RPT)],
    )


@functools.cache
def _deg_call():
    return functools.partial(
        pl.kernel,
        out_type=jax.ShapeDtypeStruct((_NC * _ACC_ROWS,), jnp.float32),
        mesh=_mesh(),
        scratch_types=[
            pltpu.VMEM((_CPT, _CH), jnp.int32),
            pltpu.VMEM((_CH,), jnp.float32),
            pltpu.VMEM((_RPT,), jnp.float32),
            pltpu.VMEM_SHARED((_ACC_ROWS,), jnp.float32),
            pltpu.SemaphoreType.DMA,
        ],
    )(_deg_body)


def _agg_body(f, y_hbm, src_hbm, dst_hbm, out_hbm, srcv, dstv, rows, acc, sems):
    cid = lax.axis_index("c")
    sid = lax.axis_index("s")
    tid = cid * _NS + sid

    def zfill(r, carry):
        for c in range(f // 16):
            rows[0][r, pl.ds(c * 16, 16)] = jnp.zeros((16,), jnp.float32)
        return carry

    lax.fori_loop(0, _ACH, zfill, 0)
    for k in range(_RPA // _ACH):
        pltpu.async_copy(rows[0], acc.at[pl.ds(sid * _RPA + k * _ACH, _ACH)],
                         sems[k % _NBUF])
    rem = _RPA % _ACH
    if rem:
        pltpu.async_copy(
            rows[0].at[pl.ds(0, rem)],
            acc.at[pl.ds(sid * _RPA + (_RPA // _ACH) * _ACH, rem)],
            sems[(_RPA // _ACH) % _NBUF],
        )
    for k in range(_RPA // _ACH):
        pltpu.make_async_copy(
            rows[0], acc.at[pl.ds(sid * _RPA + k * _ACH, _ACH)],
            sems[k % _NBUF]).wait()
    if rem:
        pltpu.make_async_copy(
            rows[0].at[pl.ds(0, rem)],
            acc.at[pl.ds(sid * _RPA + (_RPA // _ACH) * _ACH, rem)],
            sems[(_RPA // _ACH) % _NBUF]).wait()
    plsc.subcore_barrier()

    half_n = _ACPT // 4

    def stage_body(half, carry0):
        base = pl.multiple_of(tid * _ACPT + half * half_n, 8)
        pltpu.async_copy(src_hbm.at[pl.ds(base, half_n)], srcv, sems[0])
        pltpu.async_copy(dst_hbm.at[pl.ds(base, half_n)], dstv, sems[1])
        pltpu.make_async_copy(src_hbm.at[pl.ds(base, half_n)], srcv,
                              sems[0]).wait()
        pltpu.make_async_copy(dst_hbm.at[pl.ds(base, half_n)], dstv,
                              sems[1]).wait()

        for b in range(_NBUF):
            pltpu.async_copy(y_hbm.at[srcv.at[b]], rows[b], sems[b])

        def step(k, carry):
            g = k * _NBUF
            for b in range(_NBUF):
                c = g + b
                pltpu.make_async_copy(y_hbm.at[srcv.at[c]], rows[b],
                                      sems[b]).wait()
                pltpu.sync_copy(rows[b], acc.at[dstv.at[c]], add=True)

                @pl.when(c + _NBUF < half_n)
                def _():
                    pltpu.async_copy(y_hbm.at[srcv.at[c + _NBUF]], rows[b],
                                     sems[b])

            return carry

        lax.fori_loop(0, half_n // _NBUF, step, 0)
        return carry0

    lax.fori_loop(0, 4, stage_body, 0)

    plsc.subcore_barrier()
    pltpu.sync_copy(
        acc.at[pl.ds(sid * _RPA, _RPA)],
        out_hbm.at[cid, pl.ds(sid * _RPA, _RPA)],
    )


@functools.cache
def _make_agg(f):
    return functools.partial(
        pl.kernel,
        out_type=jax.ShapeDtypeStruct((_NC, _ACC_A, f), jnp.float32),
        mesh=_mesh(),
        scratch_types=[
            pltpu.VMEM((_ACPT // 4, _ACH), jnp.int32),
            pltpu.VMEM((_ACPT // 4, _ACH), jnp.int32),
            [pltpu.VMEM((_ACH, f), jnp.float32) for _ in range(_NBUF)],
            pltpu.VMEM_SHARED((_ACC_A, f), jnp.float32),
            [pltpu.SemaphoreType.DMA for _ in range(_NBUF)],
        ],
    )(functools.partial(_agg_body, f))

_BLK = 1000
_GRID = _N // _BLK


def _dense1_body(x_ref, w_ref, deg_ref, y_ref):
    dinv = lax.rsqrt(deg_ref[0] + deg_ref[1] + 1.0)
    y_ref[...] = dinv * jnp.dot(
        x_ref[...], w_ref[...], preferred_element_type=jnp.float32
    )


def _dense2_body(agg_ref, y1_ref, deg_ref, b1_ref, q_ref):
    dinv = lax.rsqrt(deg_ref[0] + deg_ref[1] + 1.0)
    s = agg_ref[0] + agg_ref[1] + y1_ref[...]
    q_ref[...] = dinv * jnp.maximum(dinv * s + b1_ref[...], 0.0)


def _dense3_body(agg_ref, q_ref, deg_ref, w2_ref, b2_ref, out_ref):
    dinv = lax.rsqrt(deg_ref[0] + deg_ref[1] + 1.0)
    s = agg_ref[0] + agg_ref[1] + q_ref[...]
    out_ref[...] = dinv * jnp.dot(
        s, w2_ref[...], preferred_element_type=jnp.float32
    ) + b2_ref[...]


def _row_spec(f):
    return pl.BlockSpec((_BLK, f), lambda i: (i, 0))


def _full_spec(shape):
    return pl.BlockSpec(shape, lambda i: tuple(0 for _ in shape))


_deg_spec = pl.BlockSpec((_NC, _BLK, 1), lambda i: (0, i, 0))
_agg_spec_h = pl.BlockSpec((_NC, _BLK, _NHID), lambda i: (0, i, 0))

_dense1 = pl.pallas_call(
    _dense1_body,
    grid=(_GRID,),
    in_specs=[_row_spec(_NFEAT), _full_spec((_NFEAT, _NHID)), _deg_spec],
    out_specs=_row_spec(_NHID),
    out_shape=jax.ShapeDtypeStruct((_N, _NHID), jnp.float32),
)

_dense2 = pl.pallas_call(
    _dense2_body,
    grid=(_GRID,),
    in_specs=[
        _agg_spec_h,
        _row_spec(_NHID),
        _deg_spec,
        _full_spec((1, _NHID)),
    ],
    out_specs=_row_spec(_NHID),
    out_shape=jax.ShapeDtypeStruct((_N, _NHID), jnp.float32),
)

_dense3 = pl.pallas_call(
    _dense3_body,
    grid=(_GRID,),
    in_specs=[
        _agg_spec_h,
        _row_spec(_NHID),
        _deg_spec,
        _full_spec((_NHID, _NCLASS)),
        _full_spec((1, _NCLASS)),
    ],
    out_specs=_row_spec(_NCLASS),
    out_shape=jax.ShapeDtypeStruct((_N, _NCLASS), jnp.float32),
)


def kernel(x, edge_index, W1, b1, W2, b2):
    src = edge_index[0]
    dst = edge_index[1]
    pad = _E_PAD - _E
    pad_idx = jnp.arange(pad, dtype=jnp.int32)
    src_f = jnp.concatenate([src, pad_idx % _N])
    dst_f = jnp.concatenate([dst, _N + (pad_idx % 16)])
    dst2d = dst_f.reshape(_CHUNKS_PAD, _CH)
    src_a = src_f.reshape(_NT * _ACPT, _ACH)
    dst_a = dst_f.reshape(_NT * _ACPT, _ACH)

    deg_p = _deg_call()(dst2d).reshape(_NC, _ACC_ROWS)
    deg_col = deg_p[:, :, None]

    y1 = _dense1(x, W1, deg_col)
    agg1 = _make_agg(_NHID)(y1, src_a, dst_a)
    q = _dense2(agg1, y1, deg_col, b1.reshape(1, _NHID))
    agg2 = _make_agg(_NHID)(q, src_a, dst_a)
    return _dense3(agg2, q, deg_col, W2, b2.reshape(1, _NCLASS))


# speedup vs baseline: 1.1269x; 1.0229x over previous
import functools

import jax
import jax.numpy as jnp
from jax import lax
from jax.experimental import pallas as pl
from jax.experimental.pallas import tpu as pltpu
from jax.experimental.pallas import tpu_sc as plsc

_N = 10000
_E = 320000
_NFEAT = 128
_NHID = 128
_NCLASS = 64

_NC = 2
_NS = 16
_NT = _NC * _NS
_CH = 128
_CPT = 80
_CHUNKS_PAD = _CPT * _NT
_E_PAD = _CHUNKS_PAD * _CH
_ACH = 64
_ACPT = _E_PAD // (_NT * _ACH)
_NBUF = 4
_ACC_ROWS = 10240
_RPT = _ACC_ROWS // _NS
_ACC_A = 10112
_RPA = _ACC_A // _NS

@functools.cache
def _mesh():
    return plsc.VectorSubcoreMesh(
        core_axis_name="c", subcore_axis_name="s", num_cores=_NC, num_subcores=_NS
    )


def _deg_body(dst_hbm, out_hbm, idx_v, ones_v, zero_v, acc, dsem):
    cid = lax.axis_index("c")
    sid = lax.axis_index("s")
    tid = cid * _NS + sid
    for i in range(_CH // 16):
        ones_v[pl.ds(i * 16, 16)] = jnp.ones((16,), jnp.float32)
    for i in range(_RPT // 16):
        zero_v[pl.ds(i * 16, 16)] = jnp.zeros((16,), jnp.float32)
    pltpu.sync_copy(zero_v, acc.at[pl.ds(sid * _RPT, _RPT)])
    plsc.subcore_barrier()
    pltpu.sync_copy(dst_hbm.at[pl.ds(tid * _CPT, _CPT)], idx_v)

    _DEPTH = 8

    def prime(i, carry):
        pltpu.async_copy(ones_v, acc.at[idx_v.at[i]], dsem, add=True)
        return carry

    lax.fori_loop(0, _DEPTH, prime, 0)

    def body(i, carry):
        pltpu.make_async_copy(ones_v, acc.at[idx_v.at[0]], dsem).wait()
        pltpu.async_copy(ones_v, acc.at[idx_v.at[i + _DEPTH]], dsem, add=True)
        return carry

    lax.fori_loop(0, _CPT - _DEPTH, body, 0)

    def drain(i, carry):
        pltpu.make_async_copy(ones_v, acc.at[idx_v.at[0]], dsem).wait()
        return carry

    lax.fori_loop(0, _DEPTH, drain, 0)
    plsc.subcore_barrier()
    pltpu.sync_copy(
        acc.at[pl.ds(sid * _RPT, _RPT)],
        out_hbm.at[pl.ds(cid * _ACC_ROWS + sid * _RPT, _RPT)],
    )


@functools.cache
def _deg_call():
    return functools.partial(
        pl.kernel,
        out_type=jax.ShapeDtypeStruct((_NC * _ACC_ROWS,), jnp.float32),
        mesh=_mesh(),
        scratch_types=[
            pltpu.VMEM((_CPT, _CH), jnp.int32),
            pltpu.VMEM((_CH,), jnp.float32),
            pltpu.VMEM((_RPT,), jnp.float32),
            pltpu.VMEM_SHARED((_ACC_ROWS,), jnp.float32),
            pltpu.SemaphoreType.DMA,
        ],
    )(_deg_body)


def _agg_body(f, y_hbm, src_hbm, dst_hbm, out_hbm, srcv, dstv, rows, acc, sems):
    cid = lax.axis_index("c")
    sid = lax.axis_index("s")
    tid = cid * _NS + sid

    def zfill(r, carry):
        for c in range(f // 16):
            rows[0][r, pl.ds(c * 16, 16)] = jnp.zeros((16,), jnp.float32)
        return carry

    lax.fori_loop(0, _ACH, zfill, 0)
    for k in range(_RPA // _ACH):
        pltpu.async_copy(rows[0], acc.at[pl.ds(sid * _RPA + k * _ACH, _ACH)],
                         sems[k % _NBUF])
    rem = _RPA % _ACH
    if rem:
        pltpu.async_copy(
            rows[0].at[pl.ds(0, rem)],
            acc.at[pl.ds(sid * _RPA + (_RPA // _ACH) * _ACH, rem)],
            sems[(_RPA // _ACH) % _NBUF],
        )
    for k in range(_RPA // _ACH):
        pltpu.make_async_copy(
            rows[0], acc.at[pl.ds(sid * _RPA + k * _ACH, _ACH)],
            sems[k % _NBUF]).wait()
    if rem:
        pltpu.make_async_copy(
            rows[0].at[pl.ds(0, rem)],
            acc.at[pl.ds(sid * _RPA + (_RPA // _ACH) * _ACH, rem)],
            sems[(_RPA // _ACH) % _NBUF]).wait()
    plsc.subcore_barrier()

    half_n = _ACPT // 4

    def stage_body(half, carry0):
        base = pl.multiple_of(tid * _ACPT + half * half_n, 8)
        pltpu.async_copy(src_hbm.at[pl.ds(base, half_n)], srcv, sems[0])
        pltpu.async_copy(dst_hbm.at[pl.ds(base, half_n)], dstv, sems[1])
        pltpu.make_async_copy(src_hbm.at[pl.ds(base, half_n)], srcv,
                              sems[0]).wait()
        pltpu.make_async_copy(dst_hbm.at[pl.ds(base, half_n)], dstv,
                              sems[1]).wait()

        for b in range(_NBUF):
            pltpu.async_copy(y_hbm.at[srcv.at[b]], rows[b], sems[b])

        def step(k, carry):
            g = k * _NBUF
            for b in range(_NBUF):
                c = g + b
                pltpu.make_async_copy(y_hbm.at[srcv.at[c]], rows[b],
                                      sems[b]).wait()
                pltpu.sync_copy(rows[b], acc.at[dstv.at[c]], add=True)

                @pl.when(c + _NBUF < half_n)
                def _():
                    pltpu.async_copy(y_hbm.at[srcv.at[c + _NBUF]], rows[b],
                                     sems[b])

            return carry

        lax.fori_loop(0, half_n // _NBUF, step, 0)
        return carry0

    lax.fori_loop(0, 4, stage_body, 0)

    plsc.subcore_barrier()
    pltpu.sync_copy(
        acc.at[pl.ds(sid * _RPA, _RPA)],
        out_hbm.at[cid, pl.ds(sid * _RPA, _RPA)],
    )


@functools.cache
def _make_agg(f):
    return functools.partial(
        pl.kernel,
        out_type=jax.ShapeDtypeStruct((_NC, _ACC_A, f), jnp.float32),
        mesh=_mesh(),
        scratch_types=[
            pltpu.VMEM((_ACPT // 4, _ACH), jnp.int32),
            pltpu.VMEM((_ACPT // 4, _ACH), jnp.int32),
            [pltpu.VMEM((_ACH, f), jnp.float32) for _ in range(_NBUF)],
            pltpu.VMEM_SHARED((_ACC_A, f), jnp.float32),
            [pltpu.SemaphoreType.DMA for _ in range(_NBUF)],
        ],
    )(functools.partial(_agg_body, f))

_BLK = 2000
_GRID = _N // _BLK


def _dense1_body(x_ref, w_ref, deg_ref, y_ref):
    dinv = lax.rsqrt(deg_ref[0] + deg_ref[1] + 1.0)
    y_ref[...] = dinv * jnp.dot(
        x_ref[...], w_ref[...], preferred_element_type=jnp.float32
    )


def _dense2_body(agg_ref, y1_ref, deg_ref, b1_ref, q_ref):
    dinv = lax.rsqrt(deg_ref[0] + deg_ref[1] + 1.0)
    s = agg_ref[0] + agg_ref[1] + y1_ref[...]
    q_ref[...] = dinv * jnp.maximum(dinv * s + b1_ref[...], 0.0)


def _dense3_body(agg_ref, q_ref, deg_ref, w2_ref, b2_ref, out_ref):
    dinv = lax.rsqrt(deg_ref[0] + deg_ref[1] + 1.0)
    s = agg_ref[0] + agg_ref[1] + q_ref[...]
    out_ref[...] = dinv * jnp.dot(
        s, w2_ref[...], preferred_element_type=jnp.float32
    ) + b2_ref[...]


def _row_spec(f):
    return pl.BlockSpec((_BLK, f), lambda i: (i, 0))


def _full_spec(shape):
    return pl.BlockSpec(shape, lambda i: tuple(0 for _ in shape))


_deg_spec = pl.BlockSpec((_NC, _BLK, 1), lambda i: (0, i, 0))
_agg_spec_h = pl.BlockSpec((_NC, _BLK, _NHID), lambda i: (0, i, 0))

_dense1 = pl.pallas_call(
    _dense1_body,
    grid=(_GRID,),
    in_specs=[_row_spec(_NFEAT), _full_spec((_NFEAT, _NHID)), _deg_spec],
    out_specs=_row_spec(_NHID),
    out_shape=jax.ShapeDtypeStruct((_N, _NHID), jnp.float32),
)

_dense2 = pl.pallas_call(
    _dense2_body,
    grid=(_GRID,),
    in_specs=[
        _agg_spec_h,
        _row_spec(_NHID),
        _deg_spec,
        _full_spec((1, _NHID)),
    ],
    out_specs=_row_spec(_NHID),
    out_shape=jax.ShapeDtypeStruct((_N, _NHID), jnp.float32),
)

_dense3 = pl.pallas_call(
    _dense3_body,
    grid=(_GRID,),
    in_specs=[
        _agg_spec_h,
        _row_spec(_NHID),
        _deg_spec,
        _full_spec((_NHID, _NCLASS)),
        _full_spec((1, _NCLASS)),
    ],
    out_specs=_row_spec(_NCLASS),
    out_shape=jax.ShapeDtypeStruct((_N, _NCLASS), jnp.float32),
)


def kernel(x, edge_index, W1, b1, W2, b2):
    src = edge_index[0]
    dst = edge_index[1]
    pad = _E_PAD - _E
    pad_idx = jnp.arange(pad, dtype=jnp.int32)
    src_f = jnp.concatenate([src, pad_idx % _N])
    dst_f = jnp.concatenate([dst, _N + (pad_idx % 16)])
    dst2d = dst_f.reshape(_CHUNKS_PAD, _CH)
    src_a = src_f.reshape(_NT * _ACPT, _ACH)
    dst_a = dst_f.reshape(_NT * _ACPT, _ACH)

    deg_p = _deg_call()(dst2d).reshape(_NC, _ACC_ROWS)
    deg_col = deg_p[:, :, None]

    y1 = _dense1(x, W1, deg_col)
    agg1 = _make_agg(_NHID)(y1, src_a, dst_a)
    q = _dense2(agg1, y1, deg_col, b1.reshape(1, _NHID))
    agg2 = _make_agg(_NHID)(q, src_a, dst_a)
    return _dense3(agg2, q, deg_col, W2, b2.reshape(1, _NCLASS))


# speedup vs baseline: 1.1306x; 1.0033x over previous
import functools

import jax
import jax.numpy as jnp
from jax import lax
from jax.experimental import pallas as pl
from jax.experimental.pallas import tpu as pltpu
from jax.experimental.pallas import tpu_sc as plsc

_N = 10000
_E = 320000
_NFEAT = 128
_NHID = 128
_NCLASS = 64

_NC = 2
_NS = 16
_NT = _NC * _NS
_CH = 128
_CPT = 80
_CHUNKS_PAD = _CPT * _NT
_E_PAD = _CHUNKS_PAD * _CH
_ACH = 64
_ACPT = _E_PAD // (_NT * _ACH)
_NBUF = 4
_ACC_ROWS = 10240
_RPT = _ACC_ROWS // _NS
_ACC_A = 10112
_RPA = _ACC_A // _NS

@functools.cache
def _mesh():
    return plsc.VectorSubcoreMesh(
        core_axis_name="c", subcore_axis_name="s", num_cores=_NC, num_subcores=_NS
    )


def _deg_body(dst_hbm, out_hbm, idx_v, ones_v, zero_v, acc, dsem):
    cid = lax.axis_index("c")
    sid = lax.axis_index("s")
    tid = cid * _NS + sid
    for i in range(_CH // 16):
        ones_v[pl.ds(i * 16, 16)] = jnp.ones((16,), jnp.float32)
    for i in range(_RPT // 16):
        zero_v[pl.ds(i * 16, 16)] = jnp.zeros((16,), jnp.float32)
    pltpu.sync_copy(zero_v, acc.at[pl.ds(sid * _RPT, _RPT)])
    plsc.subcore_barrier()
    pltpu.sync_copy(dst_hbm.at[pl.ds(tid * _CPT, _CPT)], idx_v)

    _DEPTH = 8

    def prime(i, carry):
        pltpu.async_copy(ones_v, acc.at[idx_v.at[i]], dsem, add=True)
        return carry

    lax.fori_loop(0, _DEPTH, prime, 0)

    def body(i, carry):
        pltpu.make_async_copy(ones_v, acc.at[idx_v.at[0]], dsem).wait()
        pltpu.async_copy(ones_v, acc.at[idx_v.at[i + _DEPTH]], dsem, add=True)
        return carry

    lax.fori_loop(0, _CPT - _DEPTH, body, 0)

    def drain(i, carry):
        pltpu.make_async_copy(ones_v, acc.at[idx_v.at[0]], dsem).wait()
        return carry

    lax.fori_loop(0, _DEPTH, drain, 0)
    plsc.subcore_barrier()
    pltpu.sync_copy(
        acc.at[pl.ds(sid * _RPT, _RPT)],
        out_hbm.at[pl.ds(cid * _ACC_ROWS + sid * _RPT, _RPT)],
    )


@functools.cache
def _deg_call():
    return functools.partial(
        pl.kernel,
        out_type=jax.ShapeDtypeStruct((_NC * _ACC_ROWS,), jnp.float32),
        mesh=_mesh(),
        scratch_types=[
            pltpu.VMEM((_CPT, _CH), jnp.int32),
            pltpu.VMEM((_CH,), jnp.float32),
            pltpu.VMEM((_RPT,), jnp.float32),
            pltpu.VMEM_SHARED((_ACC_ROWS,), jnp.float32),
            pltpu.SemaphoreType.DMA,
        ],
    )(_deg_body)


def _agg_body(f, y_hbm, src_hbm, dst_hbm, out_hbm, srcv, dstv, rows, acc, sems):
    cid = lax.axis_index("c")
    sid = lax.axis_index("s")
    tid = cid * _NS + sid

    def zfill(r, carry):
        for c in range(f // 16):
            rows[0][r, pl.ds(c * 16, 16)] = jnp.zeros((16,), jnp.float32)
        return carry

    lax.fori_loop(0, _ACH, zfill, 0)
    for k in range(_RPA // _ACH):
        pltpu.async_copy(rows[0], acc.at[pl.ds(sid * _RPA + k * _ACH, _ACH)],
                         sems[k % _NBUF])
    rem = _RPA % _ACH
    if rem:
        pltpu.async_copy(
            rows[0].at[pl.ds(0, rem)],
            acc.at[pl.ds(sid * _RPA + (_RPA // _ACH) * _ACH, rem)],
            sems[(_RPA // _ACH) % _NBUF],
        )
    for k in range(_RPA // _ACH):
        pltpu.make_async_copy(
            rows[0], acc.at[pl.ds(sid * _RPA + k * _ACH, _ACH)],
            sems[k % _NBUF]).wait()
    if rem:
        pltpu.make_async_copy(
            rows[0].at[pl.ds(0, rem)],
            acc.at[pl.ds(sid * _RPA + (_RPA // _ACH) * _ACH, rem)],
            sems[(_RPA // _ACH) % _NBUF]).wait()
    plsc.subcore_barrier()

    half_n = _ACPT // 4

    def stage_body(half, carry0):
        base = pl.multiple_of(tid * _ACPT + half * half_n, 8)
        pltpu.async_copy(src_hbm.at[pl.ds(base, half_n)], srcv, sems[0])
        pltpu.async_copy(dst_hbm.at[pl.ds(base, half_n)], dstv, sems[1])
        pltpu.make_async_copy(src_hbm.at[pl.ds(base, half_n)], srcv,
                              sems[0]).wait()
        pltpu.make_async_copy(dst_hbm.at[pl.ds(base, half_n)], dstv,
                              sems[1]).wait()

        for b in range(_NBUF):
            pltpu.async_copy(y_hbm.at[srcv.at[b]], rows[b], sems[b])

        def step(k, carry):
            g = k * _NBUF
            for b in range(_NBUF):
                c = g + b
                pltpu.make_async_copy(y_hbm.at[srcv.at[c]], rows[b],
                                      sems[b]).wait()
                pltpu.sync_copy(rows[b], acc.at[dstv.at[c]], add=True)

                @pl.when(c + _NBUF < half_n)
                def _():
                    pltpu.async_copy(y_hbm.at[srcv.at[c + _NBUF]], rows[b],
                                     sems[b])

            return carry

        lax.fori_loop(0, half_n // _NBUF, step, 0)
        return carry0

    lax.fori_loop(0, 4, stage_body, 0)

    plsc.subcore_barrier()
    pltpu.sync_copy(
        acc.at[pl.ds(sid * _RPA, _RPA)],
        out_hbm.at[cid, pl.ds(sid * _RPA, _RPA)],
    )


@functools.cache
def _make_agg(f):
    return functools.partial(
        pl.kernel,
        out_type=jax.ShapeDtypeStruct((_NC, _ACC_A, f), jnp.float32),
        mesh=_mesh(),
        scratch_types=[
            pltpu.VMEM((_ACPT // 4, _ACH), jnp.int32),
            pltpu.VMEM((_ACPT // 4, _ACH), jnp.int32),
            [pltpu.VMEM((_ACH, f), jnp.float32) for _ in range(_NBUF)],
            pltpu.VMEM_SHARED((_ACC_A, f), jnp.float32),
            [pltpu.SemaphoreType.DMA for _ in range(_NBUF)],
        ],
    )(functools.partial(_agg_body, f))

_BLK = 5000
_GRID = _N // _BLK


def _dense1_body(x_ref, w_ref, deg_ref, y_ref):
    dinv = lax.rsqrt(deg_ref[0] + deg_ref[1] + 1.0)
    y_ref[...] = dinv * jnp.dot(
        x_ref[...], w_ref[...], preferred_element_type=jnp.float32
    )


def _dense2_body(agg_ref, y1_ref, deg_ref, b1_ref, q_ref):
    dinv = lax.rsqrt(deg_ref[0] + deg_ref[1] + 1.0)
    s = agg_ref[0] + agg_ref[1] + y1_ref[...]
    q_ref[...] = dinv * jnp.maximum(dinv * s + b1_ref[...], 0.0)


def _dense3_body(agg_ref, q_ref, deg_ref, w2_ref, b2_ref, out_ref):
    dinv = lax.rsqrt(deg_ref[0] + deg_ref[1] + 1.0)
    s = agg_ref[0] + agg_ref[1] + q_ref[...]
    out_ref[...] = dinv * jnp.dot(
        s, w2_ref[...], preferred_element_type=jnp.float32
    ) + b2_ref[...]


def _row_spec(f):
    return pl.BlockSpec((_BLK, f), lambda i: (i, 0))


def _full_spec(shape):
    return pl.BlockSpec(shape, lambda i: tuple(0 for _ in shape))


_deg_spec = pl.BlockSpec((_NC, _BLK, 1), lambda i: (0, i, 0))
_agg_spec_h = pl.BlockSpec((_NC, _BLK, _NHID), lambda i: (0, i, 0))

_dense1 = pl.pallas_call(
    _dense1_body,
    grid=(_GRID,),
    in_specs=[_row_spec(_NFEAT), _full_spec((_NFEAT, _NHID)), _deg_spec],
    out_specs=_row_spec(_NHID),
    out_shape=jax.ShapeDtypeStruct((_N, _NHID), jnp.float32),
)

_dense2 = pl.pallas_call(
    _dense2_body,
    grid=(_GRID,),
    in_specs=[
        _agg_spec_h,
        _row_spec(_NHID),
        _deg_spec,
        _full_spec((1, _NHID)),
    ],
    out_specs=_row_spec(_NHID),
    out_shape=jax.ShapeDtypeStruct((_N, _NHID), jnp.float32),
)

_dense3 = pl.pallas_call(
    _dense3_body,
    grid=(_GRID,),
    in_specs=[
        _agg_spec_h,
        _row_spec(_NHID),
        _deg_spec,
        _full_spec((_NHID, _NCLASS)),
        _full_spec((1, _NCLASS)),
    ],
    out_specs=_row_spec(_NCLASS),
    out_shape=jax.ShapeDtypeStruct((_N, _NCLASS), jnp.float32),
)


def kernel(x, edge_index, W1, b1, W2, b2):
    src = edge_index[0]
    dst = edge_index[1]
    pad = _E_PAD - _E
    pad_idx = jnp.arange(pad, dtype=jnp.int32)
    src_f = jnp.concatenate([src, pad_idx % _N])
    dst_f = jnp.concatenate([dst, _N + (pad_idx % 16)])
    dst2d = dst_f.reshape(_CHUNKS_PAD, _CH)
    src_a = src_f.reshape(_NT * _ACPT, _ACH)
    dst_a = dst_f.reshape(_NT * _ACPT, _ACH)

    deg_p = _deg_call()(dst2d).reshape(_NC, _ACC_ROWS)
    deg_col = deg_p[:, :, None]

    y1 = _dense1(x, W1, deg_col)
    agg1 = _make_agg(_NHID)(y1, src_a, dst_a)
    q = _dense2(agg1, y1, deg_col, b1.reshape(1, _NHID))
    agg2 = _make_agg(_NHID)(q, src_a, dst_a)
    return _dense3(agg2, q, deg_col, W2, b2.reshape(1, _NCLASS))
